# Initial kernel scaffold; baseline (speedup 1.0000x reference)
import jax
import jax.numpy as jnp
from jax.experimental import pallas as pl


def kernel(feat, edge_index, etypes, W1, Wl1, b1, W2, Wl2, b2):
    raise NotImplementedError("write your pallas kernel here")


# speedup vs baseline: 22.7911x; 22.7911x over previous
import functools

import jax
import jax.numpy as jnp
from jax import lax
from jax.experimental import pallas as pl
from jax.experimental.pallas import tpu as pltpu
from jax.experimental.pallas import tpu_sc as plsc

N = 10000
E = 320000
D = 128
R = 8

NUM_TILES = 32
EPT = E // NUM_TILES
C = 80
NCH = EPT // C
STRIPE = 640
LAST_STRIPE = N - 15 * STRIPE

BN = 400
GRID = N // BN


def _tc_tables(x_parts, wpack, b, *, fuse_relu_add):
    def body(x_ref, w_ref, b_ref, t_ref, s_ref):
        if fuse_relu_add:
            x = jnp.maximum(x_ref[0] + x_ref[1], 0.0)
        else:
            x = x_ref[...]
        y = jnp.dot(x, w_ref[...], preferred_element_type=jnp.float32)
        for r in range(R):
            t_ref[r] = y[:, r * D:(r + 1) * D]
        s_ref[...] = y[:, R * D:] + b_ref[...]

    if fuse_relu_add:
        x_spec = pl.BlockSpec((2, BN, D), lambda i: (0, i, 0))
    else:
        x_spec = pl.BlockSpec((BN, D), lambda i: (i, 0))
    return pl.pallas_call(
        body,
        grid=(GRID,),
        in_specs=[
            x_spec,
            pl.BlockSpec((D, (R + 1) * D), lambda i: (0, 0)),
            pl.BlockSpec((1, D), lambda i: (0, 0)),
        ],
        out_specs=[
            pl.BlockSpec((R, BN, D), lambda i: (0, i, 0)),
            pl.BlockSpec((BN, D), lambda i: (i, 0)),
        ],
        out_shape=[
            jax.ShapeDtypeStruct((R, N, D), jnp.float32),
            jax.ShapeDtypeStruct((N, D), jnp.float32),
        ],
    )(x_parts, wpack, b)


def _tc_sum2(parts):
    def body(p_ref, o_ref):
        o_ref[...] = p_ref[0] + p_ref[1]

    return pl.pallas_call(
        body,
        grid=(GRID,),
        in_specs=[pl.BlockSpec((2, BN, D), lambda i: (0, i, 0))],
        out_specs=pl.BlockSpec((BN, D), lambda i: (i, 0)),
        out_shape=jax.ShapeDtypeStruct((N, D), jnp.float32),
    )(parts)


_SC_MESH = plsc.VectorSubcoreMesh(core_axis_name="c", subcore_axis_name="s")


@functools.partial(
    pl.kernel,
    out_type=jax.ShapeDtypeStruct((2, N, D), jnp.float32),
    mesh=_SC_MESH,
    scratch_types=[
        pltpu.VMEM((NCH, C), jnp.int32),
        pltpu.VMEM((NCH, C), jnp.int32),
        pltpu.VMEM((C, D), jnp.float32),
        pltpu.VMEM_SHARED((N, D), jnp.float32),
        pltpu.SemaphoreType.DMA,
    ],
)
def _sc_aggregate(et_hbm, src_hbm, dst_hbm, table_hbm, init_hbm, zeros_hbm,
                  out_hbm, fbuf, dbuf, rows, acc, sem):
    cid = lax.axis_index("c")
    sid = lax.axis_index("s")
    wid = cid * 16 + sid

    row0 = sid * STRIPE

    def seed(src_hbm_ref):
        @pl.when(sid < 15)
        def _():
            pltpu.sync_copy(src_hbm_ref.at[pl.ds(row0, STRIPE)],
                            acc.at[pl.ds(row0, STRIPE)])

        @pl.when(sid == 15)
        def _():
            pltpu.sync_copy(src_hbm_ref.at[pl.ds(15 * STRIPE, LAST_STRIPE)],
                            acc.at[pl.ds(15 * STRIPE, LAST_STRIPE)])

    @pl.when(cid == 0)
    def _():
        seed(init_hbm)

    @pl.when(cid != 0)
    def _():
        seed(zeros_hbm)

    plsc.subcore_barrier()

    pltpu.sync_copy(src_hbm.at[wid], fbuf)
    pltpu.sync_copy(et_hbm.at[wid], dbuf)

    def flat_body(i, carry):
        for j in range(C // 16):
            sl = pl.ds(j * 16, 16)
            fbuf[i, sl] = dbuf[i, sl] * N + fbuf[i, sl]
        return carry

    lax.fori_loop(0, NCH, flat_body, 0)
    pltpu.sync_copy(dst_hbm.at[wid], dbuf)

    def step(i, carry):
        pltpu.async_copy(table_hbm.at[fbuf.at[i]], rows, sem).wait()
        pltpu.sync_copy(rows, acc.at[dbuf.at[i]], add=True)
        return carry

    lax.fori_loop(0, NCH, step, 0)

    plsc.subcore_barrier()

    @pl.when(sid < 15)
    def _():
        pltpu.sync_copy(acc.at[pl.ds(row0, STRIPE)],
                        out_hbm.at[cid, pl.ds(row0, STRIPE)])

    @pl.when(sid == 15)
    def _():
        pltpu.sync_copy(acc.at[pl.ds(15 * STRIPE, LAST_STRIPE)],
                        out_hbm.at[cid, pl.ds(15 * STRIPE, LAST_STRIPE)])


def kernel(feat, edge_index, etypes, W1, Wl1, b1, W2, Wl2, b2):
    src = edge_index[0].reshape(NUM_TILES, NCH, C)
    dst = edge_index[1].reshape(NUM_TILES, NCH, C)
    et = etypes.reshape(NUM_TILES, NCH, C)
    zeros = jnp.zeros((N, D), jnp.float32)

    wpack1 = jnp.concatenate(
        [W1.transpose(1, 0, 2).reshape(D, R * D), Wl1], axis=1)
    wpack2 = jnp.concatenate(
        [W2.transpose(1, 0, 2).reshape(D, R * D), Wl2], axis=1)

    table1, self1 = _tc_tables(feat, wpack1, b1.reshape(1, D),
                               fuse_relu_add=False)
    p1 = _sc_aggregate(et, src, dst, table1.reshape(R * N, D), self1, zeros)
    table2, self2 = _tc_tables(p1, wpack2, b2.reshape(1, D),
                               fuse_relu_add=True)
    p2 = _sc_aggregate(et, src, dst, table2.reshape(R * N, D), self2, zeros)
    return _tc_sum2(p2)
